# baseline (speedup 1.0000x reference)
import jax
import jax.numpy as jnp
from jax.experimental import pallas as pl
from jax.experimental.pallas import tpu as pltpu

B, N, D, E = 16, 512, 32, 4
BB = 4


def _ggnn_kernel(xT_ref, edges_ref, WeT_ref, WzT_ref, UzT_ref, bzT_ref,
                 WrT_ref, UrT_ref, brT_ref, WhT_ref, UhT_ref, bhT_ref,
                 WoT_ref, boT_ref, out_ref):
    for bb in range(BB):
        xT = xT_ref[bb]

        mT = jnp.zeros((D, N), dtype=jnp.float32)
        for e in range(E):
            txT = jnp.dot(WeT_ref[e], xT, preferred_element_type=jnp.float32)
            mT = mT + jax.lax.dot_general(
                txT, edges_ref[bb, e],
                dimension_numbers=(((1,), (1,)), ((), ())),
                precision=jax.lax.Precision.DEFAULT,
                preferred_element_type=jnp.float32)

        z = jax.nn.sigmoid(jnp.dot(WzT_ref[...], mT)
                           + jnp.dot(UzT_ref[...], xT) + bzT_ref[...])
        r = jax.nn.sigmoid(jnp.dot(WrT_ref[...], mT)
                           + jnp.dot(UrT_ref[...], xT) + brT_ref[...])
        h_til = jnp.tanh(jnp.dot(WhT_ref[...], mT)
                         + jnp.dot(UhT_ref[...], r * xT) + bhT_ref[...])
        hT = (1.0 - z) * xT + z * h_til

        logits = jnp.dot(WoT_ref[...], hT) + boT_ref[...]
        lmax = jnp.max(logits, axis=0, keepdims=True)
        shifted = logits - lmax
        lse = jnp.log(jnp.sum(jnp.exp(shifted), axis=0, keepdims=True))
        out_ref[bb] = shifted - lse


@jax.jit
def kernel(x_padded, x_lengths, edges, fingers, W_edge, Wz, Uz, bz,
           Wr, Ur, br, Wh, Uh, bh, W_out, b_out):
    del x_lengths, fingers
    full = lambda g: (0, 0)
    outT = pl.pallas_call(
        _ggnn_kernel,
        grid=(B // BB,),
        in_specs=[
            pl.BlockSpec((BB, D, N), lambda g: (g, 0, 0)),
            pl.BlockSpec((BB, E, N, N), lambda g: (g, 0, 0, 0)),
            pl.BlockSpec((E, D, D), lambda g: (0, 0, 0)),
            pl.BlockSpec((D, D), full),
            pl.BlockSpec((D, D), full),
            pl.BlockSpec((D, 1), full),
            pl.BlockSpec((D, D), full),
            pl.BlockSpec((D, D), full),
            pl.BlockSpec((D, 1), full),
            pl.BlockSpec((D, D), full),
            pl.BlockSpec((D, D), full),
            pl.BlockSpec((D, 1), full),
            pl.BlockSpec((5, D), full),
            pl.BlockSpec((5, 1), full),
        ],
        out_specs=pl.BlockSpec((BB, 5, N), lambda g: (g, 0, 0)),
        out_shape=jax.ShapeDtypeStruct((B, 5, N), jnp.float32),
        compiler_params=pltpu.CompilerParams(
            dimension_semantics=("arbitrary",)),
    )(x_padded.transpose(0, 2, 1), edges,
      W_edge.transpose(0, 2, 1),
      Wz.T, Uz.T, bz.reshape(D, 1),
      Wr.T, Ur.T, br.reshape(D, 1),
      Wh.T, Uh.T, bh.reshape(D, 1),
      W_out.T, b_out.reshape(5, 1))
    return outT.transpose(0, 2, 1)
